# Initial kernel scaffold; baseline (speedup 1.0000x reference)
import jax
import jax.numpy as jnp
from jax.experimental import pallas as pl


def kernel(x, logits):
    raise NotImplementedError("write your pallas kernel here")


# speedup vs baseline: 1.0366x; 1.0366x over previous
import functools

import jax
import jax.numpy as jnp
from jax.experimental import pallas as pl
from jax.experimental.pallas import tpu as pltpu

INPUT_DIM = 4096
K = 1024
TEMP = 5.0
ROWS = 4 * 2048
BLK = 512


def _mask_mul_kernel(x_ref, gl_ref, o_ref, kth_smem):
    sp = jax.nn.sigmoid(gl_ref[...] * (1.0 / TEMP))

    @pl.when(pl.program_id(0) == 0)
    def _find_kth():
        def body(_, carry):
            lo, hi = carry
            mid = lo + (hi - lo + 1) // 2
            cnt = jnp.sum(
                (sp >= jax.lax.bitcast_convert_type(mid, jnp.float32)).astype(
                    jnp.int32
                )
            )
            big = cnt >= K
            return (jnp.where(big, mid, lo), jnp.where(big, hi, mid - 1))

        lo = jnp.int32(0)
        hi = jnp.int32(0x3F800000)
        lo, hi = jax.lax.fori_loop(0, 31, body, (lo, hi))
        kth_smem[0] = lo

    kth = jax.lax.bitcast_convert_type(kth_smem[0], jnp.float32)
    mask = (sp >= kth).astype(jnp.float32)
    o_ref[...] = x_ref[...] * mask


@jax.jit
def kernel(x, logits):
    u = jnp.clip(
        jax.random.uniform(jax.random.key(1), logits.shape, dtype=jnp.float32),
        1e-06,
        None,
    )
    noise = -jnp.log(-jnp.log(u) + 1e-06)
    gl = (logits + noise).reshape(1, INPUT_DIM)

    x2d = x.reshape(ROWS, INPUT_DIM)
    out = pl.pallas_call(
        _mask_mul_kernel,
        grid=(ROWS // BLK,),
        in_specs=[
            pl.BlockSpec((BLK, INPUT_DIM), lambda i: (i, 0)),
            pl.BlockSpec((1, INPUT_DIM), lambda i: (0, 0)),
        ],
        out_specs=pl.BlockSpec((BLK, INPUT_DIM), lambda i: (i, 0)),
        out_shape=jax.ShapeDtypeStruct((ROWS, INPUT_DIM), jnp.float32),
        scratch_shapes=[pltpu.SMEM((1,), jnp.int32)],
        compiler_params=pltpu.CompilerParams(
            dimension_semantics=("arbitrary",),
        ),
    )(x2d, gl)
    return out.reshape(x.shape)
